# Initial kernel scaffold; baseline (speedup 1.0000x reference)
import jax
import jax.numpy as jnp
from jax.experimental import pallas as pl


def kernel(feats, batch):
    raise NotImplementedError("write your pallas kernel here")


# speedup vs baseline: 2.0091x; 2.0091x over previous
import functools

import jax
import jax.numpy as jnp
from jax import lax
from jax.experimental import pallas as pl
from jax.experimental.pallas import tpu as pltpu
from jax.experimental.pallas import tpu_sc as plsc

_NUM_SEGMENTS = 10000
_N = 320000
_D = 128
_NW = 32
_SPW = 320
_SPAD = _NW * _SPW
_CHUNK = 256
_NEG = float("-inf")


def _make_seg_max():
    mesh = plsc.VectorSubcoreMesh(core_axis_name="c", subcore_axis_name="s")

    @functools.partial(
        pl.kernel,
        mesh=mesh,
        out_type=jax.ShapeDtypeStruct((_SPAD, _D), jnp.float32),
        scratch_types=[
            pltpu.VMEM((48,), jnp.int32),
            pltpu.VMEM((_SPW + 1, _D), jnp.float32),
            pltpu.VMEM((_CHUNK, _D), jnp.float32),
            pltpu.VMEM((_CHUNK,), jnp.int32),
        ],
    )
    def seg_max(feats_hbm, batch_hbm, bounds_hbm, out_hbm, bounds_v, acc,
                rows, ids):
        wid = lax.axis_index("s") * 2 + lax.axis_index("c")
        seg_base = wid * _SPW

        pltpu.sync_copy(bounds_hbm, bounds_v)
        bv = bounds_v[pl.ds(wid, 16)]
        row_lo = bv[0]
        row_hi = bv[1]
        row_lo_a = (row_lo // 8) * 8
        nchunks = (row_hi - row_lo_a + _CHUNK - 1) // _CHUNK

        neg = jnp.full((16,), _NEG, jnp.float32)

        def init_body(i, _):
            r = i // 8
            k = i % 8
            acc[r, pl.ds(k * 16, 16)] = neg
            return 0

        lax.fori_loop(0, (_SPW + 1) * 8, init_body, 0)

        def chunk_body(c, _):
            start = jnp.minimum(row_lo_a + c * _CHUNK, _N - _CHUNK)
            pltpu.sync_copy(feats_hbm.at[pl.ds(start, _CHUNK)], rows)
            pltpu.sync_copy(batch_hbm.at[pl.ds(start, _CHUNK)], ids)

            def grp_body(g, _):
                idv = ids[pl.ds(g * 16, 16)] - seg_base
                bad = (idv < 0) | (idv >= _SPW)
                slv = jnp.where(bad, _SPW, idv)
                for j in range(16):
                    sl = slv[j]
                    r = g * 16 + j
                    for k in range(8):
                        v = rows[r, pl.ds(k * 16, 16)]
                        a = acc[sl, pl.ds(k * 16, 16)]
                        acc[sl, pl.ds(k * 16, 16)] = jnp.maximum(a, v)
                return 0

            lax.fori_loop(0, _CHUNK // 16, grp_body, 0)
            return 0

        lax.fori_loop(0, nchunks, chunk_body, 0)

        def fix_body(i, _):
            r = i // 8
            k = i % 8
            a = acc[r, pl.ds(k * 16, 16)]
            acc[r, pl.ds(k * 16, 16)] = jnp.where(a == _NEG, 0.0, a)
            return 0

        lax.fori_loop(0, _SPW * 8, fix_body, 0)
        pltpu.sync_copy(acc.at[pl.ds(0, _SPW)],
                        out_hbm.at[pl.ds(seg_base, _SPW)])

    return seg_max


_seg_max = _make_seg_max()


@jax.jit
def kernel(feats, batch):
    targets = jnp.arange(_NW + 1, dtype=jnp.int32) * _SPW
    bounds = jnp.searchsorted(batch, targets, side="left").astype(jnp.int32)
    bounds = jnp.pad(bounds, (0, 48 - (_NW + 1)))
    out = _seg_max(feats, batch, bounds)
    return out[:_NUM_SEGMENTS]


# speedup vs baseline: 3.2704x; 1.6278x over previous
import functools

import jax
import jax.numpy as jnp
from jax import lax
from jax.experimental import pallas as pl
from jax.experimental.pallas import tpu as pltpu
from jax.experimental.pallas import tpu_sc as plsc

_NUM_SEGMENTS = 10000
_N = 320000
_D = 128
_NW = 32
_SPW = 320
_SPAD = _NW * _SPW
_CHUNK = 256
_NEG = float("-inf")


def _make_seg_max():
    mesh = plsc.VectorSubcoreMesh(core_axis_name="c", subcore_axis_name="s")

    @functools.partial(
        pl.kernel,
        mesh=mesh,
        out_type=jax.ShapeDtypeStruct((_SPAD, _D), jnp.float32),
        scratch_types=[
            pltpu.VMEM((48,), jnp.int32),
            pltpu.VMEM((_SPW + 1, _D), jnp.float32),
            pltpu.VMEM((_CHUNK, _D), jnp.float32),
            pltpu.VMEM((_CHUNK,), jnp.int32),
        ],
    )
    def seg_max(feats_hbm, batch_hbm, bounds_hbm, out_hbm, bounds_v, acc,
                rows, ids):
        wid = lax.axis_index("s") * 2 + lax.axis_index("c")
        seg_base = wid * _SPW

        pltpu.sync_copy(bounds_hbm, bounds_v)
        bv = bounds_v[pl.ds(wid, 16)]
        row_lo = bv[0]
        row_hi = bv[1]
        row_lo_a = (row_lo // 8) * 8
        nchunks = (row_hi - row_lo_a + _CHUNK - 1) // _CHUNK

        neg = jnp.full((16,), _NEG, jnp.float32)

        def init_body(i, _):
            r = i // 8
            k = i % 8
            acc[r, pl.ds(k * 16, 16)] = neg
            return 0

        lax.fori_loop(0, (_SPW + 1) * 8, init_body, 0)

        def flush(cur, vecs):
            for k in range(8):
                a = acc[cur, pl.ds(k * 16, 16)]
                acc[cur, pl.ds(k * 16, 16)] = jnp.maximum(a, vecs[k])

        def chunk_body(c, carry):
            start = jnp.minimum(row_lo_a + c * _CHUNK, _N - _CHUNK)
            pltpu.sync_copy(feats_hbm.at[pl.ds(start, _CHUNK)], rows)
            pltpu.sync_copy(batch_hbm.at[pl.ds(start, _CHUNK)], ids)

            def grp_body(g, carry):
                idv = ids[pl.ds(g * 16, 16)] - seg_base
                bad = (idv < 0) | (idv >= _SPW)
                slv = jnp.where(bad, _SPW, idv)
                for j in range(16):
                    sid = slv[j]
                    r = g * 16 + j
                    rv = tuple(rows[r, pl.ds(k * 16, 16)] for k in range(8))
                    cur = carry[0]
                    same = sid == cur

                    @pl.when(jnp.logical_not(same))
                    def _(cur=cur, vecs=carry[1:]):
                        flush(cur, vecs)

                    vecs = tuple(
                        jnp.where(same, jnp.maximum(carry[k + 1], rv[k]),
                                  rv[k])
                        for k in range(8))
                    carry = (sid,) + vecs
                return carry

            return lax.fori_loop(0, _CHUNK // 16, grp_body, carry)

        carry0 = (jnp.int32(_SPW),) + tuple(neg for _ in range(8))
        carry = lax.fori_loop(0, nchunks, chunk_body, carry0)
        flush(carry[0], carry[1:])

        def fix_body(i, _):
            r = i // 8
            k = i % 8
            a = acc[r, pl.ds(k * 16, 16)]
            acc[r, pl.ds(k * 16, 16)] = jnp.where(a == _NEG, 0.0, a)
            return 0

        lax.fori_loop(0, _SPW * 8, fix_body, 0)
        pltpu.sync_copy(acc.at[pl.ds(0, _SPW)],
                        out_hbm.at[pl.ds(seg_base, _SPW)])

    return seg_max


_seg_max = _make_seg_max()


@jax.jit
def kernel(feats, batch):
    targets = jnp.arange(_NW + 1, dtype=jnp.int32) * _SPW
    bounds = jnp.searchsorted(batch, targets, side="left").astype(jnp.int32)
    bounds = jnp.pad(bounds, (0, 48 - (_NW + 1)))
    out = _seg_max(feats, batch, bounds)
    return out[:_NUM_SEGMENTS]


# speedup vs baseline: 4.4645x; 1.3651x over previous
import functools

import jax
import jax.numpy as jnp
from jax import lax
from jax.experimental import pallas as pl
from jax.experimental.pallas import tpu as pltpu
from jax.experimental.pallas import tpu_sc as plsc

_NUM_SEGMENTS = 10000
_N = 320000
_D = 128
_NW = 32
_SPW = 320
_SPAD = _NW * _SPW
_CHUNK = 256
_NEG = float("-inf")


def _make_seg_max():
    mesh = plsc.VectorSubcoreMesh(core_axis_name="c", subcore_axis_name="s")

    @functools.partial(
        pl.kernel,
        mesh=mesh,
        out_type=jax.ShapeDtypeStruct((_SPAD, _D), jnp.float32),
        scratch_types=[
            pltpu.VMEM((48,), jnp.int32),
            pltpu.VMEM((_SPW + 1, _D), jnp.float32),
            pltpu.VMEM((_CHUNK, _D), jnp.float32),
            pltpu.VMEM((_CHUNK, _D), jnp.float32),
            pltpu.VMEM((_CHUNK,), jnp.int32),
            pltpu.VMEM((_CHUNK,), jnp.int32),
            pltpu.SemaphoreType.DMA,
            pltpu.SemaphoreType.DMA,
            pltpu.SemaphoreType.DMA,
            pltpu.SemaphoreType.DMA,
        ],
    )
    def seg_max(feats_hbm, batch_hbm, bounds_hbm, out_hbm, bounds_v, acc,
                rows0, rows1, ids0, ids1, sr0, sr1, si0, si1):
        wid = lax.axis_index("s") * 2 + lax.axis_index("c")
        seg_base = wid * _SPW

        pltpu.sync_copy(bounds_hbm, bounds_v)
        bv = bounds_v[pl.ds(wid, 16)]
        row_lo = bv[0]
        row_hi = bv[1]
        row_lo_a = (row_lo // 8) * 8
        nchunks = (row_hi - row_lo_a + _CHUNK - 1) // _CHUNK

        neg = jnp.full((16,), _NEG, jnp.float32)

        def init_body(i, _):
            r = i // 8
            k = i % 8
            acc[r, pl.ds(k * 16, 16)] = neg
            return 0

        lax.fori_loop(0, (_SPW + 1) * 8, init_body, 0)

        def flush(cur, vecs):
            for k in range(8):
                a = acc[cur, pl.ds(k * 16, 16)]
                acc[cur, pl.ds(k * 16, 16)] = jnp.maximum(a, vecs[k])

        def process(rows, ids, carry):
            def grp_body(g, carry):
                idv = ids[pl.ds(g * 16, 16)] - seg_base
                bad = (idv < 0) | (idv >= _SPW)
                slv = jnp.where(bad, _SPW, idv)
                for j in range(16):
                    sid = slv[j]
                    r = g * 16 + j
                    rv = tuple(rows[r, pl.ds(k * 16, 16)] for k in range(8))
                    cur = carry[0]
                    same = sid == cur

                    @pl.when(jnp.logical_not(same))
                    def _(cur=cur, vecs=carry[1:]):
                        flush(cur, vecs)

                    vecs = tuple(
                        jnp.where(same, jnp.maximum(carry[k + 1], rv[k]),
                                  rv[k])
                        for k in range(8))
                    carry = (sid,) + vecs
                return carry

            return lax.fori_loop(0, _CHUNK // 16, grp_body, carry)

        def issue(c, rows, ids, sr, si):
            start = jnp.minimum(row_lo_a + c * _CHUNK, _N - _CHUNK)
            pltpu.async_copy(feats_hbm.at[pl.ds(start, _CHUNK)], rows, sr)
            pltpu.async_copy(batch_hbm.at[pl.ds(start, _CHUNK)], ids, si)

        def drain(rows, ids, sr, si):
            pltpu.make_async_copy(
                feats_hbm.at[pl.ds(0, _CHUNK)], rows, sr).wait()
            pltpu.make_async_copy(
                batch_hbm.at[pl.ds(0, _CHUNK)], ids, si).wait()

        nch2 = jnp.maximum(((nchunks + 1) // 2) * 2, 2)
        issue(0, rows0, ids0, sr0, si0)
        issue(1, rows1, ids1, sr1, si1)

        def super_body(s, carry):
            drain(rows0, ids0, sr0, si0)
            carry = process(rows0, ids0, carry)

            @pl.when(2 * s + 2 < nch2)
            def _():
                issue(2 * s + 2, rows0, ids0, sr0, si0)

            drain(rows1, ids1, sr1, si1)
            carry = process(rows1, ids1, carry)

            @pl.when(2 * s + 3 < nch2)
            def _():
                issue(2 * s + 3, rows1, ids1, sr1, si1)

            return carry

        carry0 = (jnp.int32(_SPW),) + tuple(neg for _ in range(8))
        carry = lax.fori_loop(0, nch2 // 2, super_body, carry0)
        flush(carry[0], carry[1:])

        def fix_body(i, _):
            r = i // 8
            k = i % 8
            a = acc[r, pl.ds(k * 16, 16)]
            acc[r, pl.ds(k * 16, 16)] = jnp.where(a == _NEG, 0.0, a)
            return 0

        lax.fori_loop(0, _SPW * 8, fix_body, 0)
        pltpu.sync_copy(acc.at[pl.ds(0, _SPW)],
                        out_hbm.at[pl.ds(seg_base, _SPW)])

    return seg_max


_seg_max = _make_seg_max()


@jax.jit
def kernel(feats, batch):
    targets = jnp.arange(_NW + 1, dtype=jnp.int32) * _SPW
    bounds = jnp.searchsorted(batch, targets, side="left").astype(jnp.int32)
    bounds = jnp.pad(bounds, (0, 48 - (_NW + 1)))
    out = _seg_max(feats, batch, bounds)
    return out[:_NUM_SEGMENTS]


# speedup vs baseline: 4.8192x; 1.0795x over previous
import functools

import jax
import jax.numpy as jnp
from jax import lax
from jax.experimental import pallas as pl
from jax.experimental.pallas import tpu as pltpu
from jax.experimental.pallas import tpu_sc as plsc

_NUM_SEGMENTS = 10000
_N = 320000
_D = 128
_NW = 32
_SPW = 320
_SPAD = _NW * _SPW
_CHUNK = 256
_NEG = float("-inf")


def _make_seg_max():
    mesh = plsc.VectorSubcoreMesh(core_axis_name="c", subcore_axis_name="s")

    @functools.partial(
        pl.kernel,
        mesh=mesh,
        out_type=jax.ShapeDtypeStruct((_SPAD, _D), jnp.float32),
        scratch_types=[
            pltpu.VMEM((48,), jnp.int32),
            pltpu.VMEM((_SPW + 1, _D), jnp.float32),
            pltpu.VMEM((_CHUNK, _D), jnp.float32),
            pltpu.VMEM((_CHUNK, _D), jnp.float32),
            pltpu.VMEM((_CHUNK,), jnp.int32),
            pltpu.VMEM((_CHUNK,), jnp.int32),
            pltpu.VMEM((_D,), jnp.float32),
            pltpu.SemaphoreType.DMA,
            pltpu.SemaphoreType.DMA,
            pltpu.SemaphoreType.DMA,
            pltpu.SemaphoreType.DMA,
        ],
    )
    def seg_max(feats_hbm, batch_hbm, bounds_hbm, out_hbm, bounds_v, acc,
                rows0, rows1, ids0, ids1, runbuf, sr0, sr1, si0, si1):
        wid = lax.axis_index("s") * 2 + lax.axis_index("c")
        seg_base = wid * _SPW

        pltpu.sync_copy(bounds_hbm, bounds_v)
        bv = bounds_v[pl.ds(wid, 16)]
        row_lo = bv[0]
        row_hi = bv[1]
        row_lo_a = (row_lo // 8) * 8
        nchunks = (row_hi - row_lo_a + _CHUNK - 1) // _CHUNK

        neg = jnp.full((16,), _NEG, jnp.float32)

        def init_body(i, _):
            r = i // 8
            k = i % 8
            acc[r, pl.ds(k * 16, 16)] = neg
            return 0

        lax.fori_loop(0, (_SPW + 1) * 8, init_body, 0)

        def flush(cur, vecs):
            for k in range(8):
                a = acc[cur, pl.ds(k * 16, 16)]
                acc[cur, pl.ds(k * 16, 16)] = jnp.maximum(a, vecs[k])

        def process(rows, ids, cur):
            def grp_body(g, cur):
                idv = ids[pl.ds(g * 16, 16)] - seg_base
                bad = (idv < 0) | (idv >= _SPW)
                slv = jnp.where(bad, _SPW, idv)
                uniform = (slv[0] == cur) & (slv[15] == cur)

                def fast(cur):
                    for k in range(8):
                        vs = [rows[g * 16 + r, pl.ds(k * 16, 16)]
                              for r in range(16)]
                        while len(vs) > 1:
                            vs = [jnp.maximum(vs[i], vs[i + 1])
                                  for i in range(0, len(vs), 2)]
                        a = runbuf[pl.ds(k * 16, 16)]
                        runbuf[pl.ds(k * 16, 16)] = jnp.maximum(a, vs[0])
                    return cur

                def slow(cur):
                    carry = (cur,) + tuple(
                        runbuf[pl.ds(k * 16, 16)] for k in range(8))
                    for j in range(16):
                        sid = slv[j]
                        rv = tuple(rows[g * 16 + j, pl.ds(k * 16, 16)]
                                   for k in range(8))
                        cur_j = carry[0]
                        same = sid == cur_j

                        @pl.when(jnp.logical_not(same))
                        def _(cur_j=cur_j, vecs=carry[1:]):
                            flush(cur_j, vecs)

                        vecs = tuple(
                            jnp.where(same, jnp.maximum(carry[k + 1], rv[k]),
                                      rv[k])
                            for k in range(8))
                        carry = (sid,) + vecs
                    for k in range(8):
                        runbuf[pl.ds(k * 16, 16)] = carry[k + 1]
                    return carry[0]

                return lax.cond(uniform, fast, slow, cur)

            return lax.fori_loop(0, _CHUNK // 16, grp_body, cur)

        def issue(c, rows, ids, sr, si):
            start = jnp.minimum(row_lo_a + c * _CHUNK, _N - _CHUNK)
            pltpu.async_copy(feats_hbm.at[pl.ds(start, _CHUNK)], rows, sr)
            pltpu.async_copy(batch_hbm.at[pl.ds(start, _CHUNK)], ids, si)

        def drain(rows, ids, sr, si):
            pltpu.make_async_copy(
                feats_hbm.at[pl.ds(0, _CHUNK)], rows, sr).wait()
            pltpu.make_async_copy(
                batch_hbm.at[pl.ds(0, _CHUNK)], ids, si).wait()

        nch2 = jnp.maximum(((nchunks + 1) // 2) * 2, 2)
        issue(0, rows0, ids0, sr0, si0)
        issue(1, rows1, ids1, sr1, si1)

        for k in range(8):
            runbuf[pl.ds(k * 16, 16)] = neg

        def super_body(s, cur):
            drain(rows0, ids0, sr0, si0)
            cur = process(rows0, ids0, cur)

            @pl.when(2 * s + 2 < nch2)
            def _():
                issue(2 * s + 2, rows0, ids0, sr0, si0)

            drain(rows1, ids1, sr1, si1)
            cur = process(rows1, ids1, cur)

            @pl.when(2 * s + 3 < nch2)
            def _():
                issue(2 * s + 3, rows1, ids1, sr1, si1)

            return cur

        cur = lax.fori_loop(0, nch2 // 2, super_body, jnp.int32(_SPW))
        flush(cur, tuple(runbuf[pl.ds(k * 16, 16)] for k in range(8)))

        def fix_body(i, _):
            r = i // 8
            k = i % 8
            a = acc[r, pl.ds(k * 16, 16)]
            acc[r, pl.ds(k * 16, 16)] = jnp.where(a == _NEG, 0.0, a)
            return 0

        lax.fori_loop(0, _SPW * 8, fix_body, 0)
        pltpu.sync_copy(acc.at[pl.ds(0, _SPW)],
                        out_hbm.at[pl.ds(seg_base, _SPW)])

    return seg_max


_seg_max = _make_seg_max()


@jax.jit
def kernel(feats, batch):
    targets = jnp.arange(_NW + 1, dtype=jnp.int32) * _SPW
    bounds = jnp.searchsorted(batch, targets, side="left").astype(jnp.int32)
    bounds = jnp.pad(bounds, (0, 48 - (_NW + 1)))
    out = _seg_max(feats, batch, bounds)
    return out[:_NUM_SEGMENTS]
